# Initial kernel scaffold; baseline (speedup 1.0000x reference)
import jax
import jax.numpy as jnp
from jax.experimental import pallas as pl


def kernel(tags, categories, tag_table, cat_table):
    raise NotImplementedError("write your pallas kernel here")


# speedup vs baseline: 6.4031x; 6.4031x over previous
import functools

import jax
import jax.numpy as jnp
from jax import lax
from jax.experimental import pallas as pl
from jax.experimental.pallas import tpu as pltpu
from jax.experimental.pallas import tpu_sc as plsc

B = 4096
D = 64
TAG_LEN = 50
CAT_LEN = 20
L = 16
NC = 2
NS = 16
NW = NC * NS
BPW = B // NW

TBAGS = 2
CBAGS = 4
TG = BPW // TBAGS
CG = BPW // CBAGS

_mesh = plsc.VectorSubcoreMesh(core_axis_name="c", subcore_axis_name="s")


@functools.partial(
    pl.kernel,
    mesh=_mesh,
    compiler_params=pltpu.CompilerParams(use_tc_tiling_on_sc=False),
    out_type=(
        jax.ShapeDtypeStruct((B, D), jnp.float32),
        jax.ShapeDtypeStruct((B, D), jnp.float32),
    ),
    scratch_types=[
        pltpu.VMEM((TG, TBAGS * TAG_LEN), jnp.int32),
        pltpu.VMEM((CG, CBAGS * CAT_LEN), jnp.int32),
        pltpu.VMEM((TBAGS * TAG_LEN, D), jnp.float32),
        pltpu.VMEM((BPW, D), jnp.float32),
        pltpu.VMEM((BPW, D), jnp.float32),
        pltpu.SemaphoreType.DMA,
    ],
)
def _encode(tags_g, cats_g, tag_tab, cat_tab, out_t, out_c,
            tidx, cidx, rows, acc_t, acc_c, sem):
    wid = lax.axis_index("s") * NC + lax.axis_index("c")
    b_base = wid * BPW

    pltpu.sync_copy(tags_g.at[pl.ds(wid * TG, TG)], tidx)
    pltpu.sync_copy(cats_g.at[pl.ds(wid * CG, CG)], cidx)

    def reduce_group(g, table_rows, n_bags, bag_len, acc):
        for q in range(n_bags):
            b = g * n_bags + q
            for d in range(4):
                sl = pl.ds(d * L, L)
                v = table_rows[q * bag_len, sl]
                for j in range(1, bag_len):
                    v = v + table_rows[q * bag_len + j, sl]
                acc[b, sl] = v

    def t_body(g, carry):
        pltpu.async_copy(tag_tab.at[tidx.at[g]], rows, sem).wait()
        reduce_group(g, rows, TBAGS, TAG_LEN, acc_t)
        return carry

    lax.fori_loop(0, TG, t_body, 0)

    crows = rows.at[pl.ds(0, CBAGS * CAT_LEN)]

    def c_body(g, carry):
        pltpu.async_copy(cat_tab.at[cidx.at[g]], crows, sem).wait()
        reduce_group(g, crows, CBAGS, CAT_LEN, acc_c)
        return carry

    lax.fori_loop(0, CG, c_body, 0)

    pltpu.sync_copy(acc_t, out_t.at[pl.ds(b_base, BPW)])
    pltpu.sync_copy(acc_c, out_c.at[pl.ds(b_base, BPW)])


def kernel(tags, categories, tag_table, cat_table):
    tags_g = tags.reshape(B // TBAGS, TBAGS * TAG_LEN)
    cats_g = categories.reshape(B // CBAGS, CBAGS * CAT_LEN)
    return _encode(tags_g, cats_g, tag_table, cat_table)


# speedup vs baseline: 9.3194x; 1.4554x over previous
import functools

import jax
import jax.numpy as jnp
from jax import lax
from jax.experimental import pallas as pl
from jax.experimental.pallas import tpu as pltpu
from jax.experimental.pallas import tpu_sc as plsc

B = 4096
D = 64
TAG_LEN = 50
CAT_LEN = 20
L = 16
NC = 2
NS = 16
NW = NC * NS
BPW = B // NW

TBAGS = 2
CBAGS = 4
TG = BPW // TBAGS
CG = BPW // CBAGS

_mesh = plsc.VectorSubcoreMesh(core_axis_name="c", subcore_axis_name="s")


@functools.partial(
    pl.kernel,
    mesh=_mesh,
    compiler_params=pltpu.CompilerParams(use_tc_tiling_on_sc=False),
    out_type=(
        jax.ShapeDtypeStruct((B, D), jnp.float32),
        jax.ShapeDtypeStruct((B, D), jnp.float32),
    ),
    scratch_types=[
        pltpu.VMEM((TG, TBAGS * TAG_LEN), jnp.int32),
        pltpu.VMEM((CG, CBAGS * CAT_LEN), jnp.int32),
        pltpu.VMEM((2, TBAGS * TAG_LEN, D), jnp.float32),
        pltpu.VMEM((BPW, D), jnp.float32),
        pltpu.VMEM((BPW, D), jnp.float32),
        pltpu.SemaphoreType.DMA,
        pltpu.SemaphoreType.DMA,
    ],
)
def _encode(tags_g, cats_g, tag_tab, cat_tab, out_t, out_c,
            tidx, cidx, rows, acc_t, acc_c, sem0, sem1):
    wid = lax.axis_index("s") * NC + lax.axis_index("c")
    b_base = wid * BPW
    sems = (sem0, sem1)

    pltpu.sync_copy(tags_g.at[pl.ds(wid * TG, TG)], tidx)
    pltpu.sync_copy(cats_g.at[pl.ds(wid * CG, CG)], cidx)

    def reduce_group(p, g, n_bags, bag_len, acc):
        for q in range(n_bags):
            b = g * n_bags + q
            r0 = q * bag_len
            for d in range(4):
                sl = pl.ds(d * L, L)
                v0 = rows[p, r0, sl]
                v1 = rows[p, r0 + 1, sl]
                for j in range(2, bag_len, 2):
                    v0 = v0 + rows[p, r0 + j, sl]
                    v1 = v1 + rows[p, r0 + j + 1, sl]
                acc[b, sl] = v0 + v1

    def t_fire(g, p):
        pltpu.async_copy(tag_tab.at[tidx.at[g]], rows.at[p], sems[p])

    def t_wait(p):
        pltpu.make_async_copy(tag_tab.at[tidx.at[0]], rows.at[p], sems[p]).wait()

    t_fire(0, 0)
    t_fire(1, 1)

    def t_body(gg, carry):
        for p in range(2):
            g = 2 * gg + p
            t_wait(p)
            reduce_group(p, g, TBAGS, TAG_LEN, acc_t)

            @pl.when(g + 2 < TG)
            def _():
                t_fire(g + 2, p)
        return carry

    lax.fori_loop(0, TG // 2, t_body, 0)

    def c_fire(g, p):
        pltpu.async_copy(cat_tab.at[cidx.at[g]],
                         rows.at[p, pl.ds(0, CBAGS * CAT_LEN)], sems[p])

    def c_wait(p):
        pltpu.make_async_copy(cat_tab.at[cidx.at[0]],
                              rows.at[p, pl.ds(0, CBAGS * CAT_LEN)],
                              sems[p]).wait()

    c_fire(0, 0)
    c_fire(1, 1)

    def c_body(gg, carry):
        for p in range(2):
            g = 2 * gg + p
            c_wait(p)
            reduce_group(p, g, CBAGS, CAT_LEN, acc_c)

            @pl.when(g + 2 < CG)
            def _():
                c_fire(g + 2, p)
        return carry

    lax.fori_loop(0, CG // 2, c_body, 0)

    pltpu.sync_copy(acc_t, out_t.at[pl.ds(b_base, BPW)])
    pltpu.sync_copy(acc_c, out_c.at[pl.ds(b_base, BPW)])


def kernel(tags, categories, tag_table, cat_table):
    tags_g = tags.reshape(B // TBAGS, TBAGS * TAG_LEN)
    cats_g = categories.reshape(B // CBAGS, CBAGS * CAT_LEN)
    return _encode(tags_g, cats_g, tag_table, cat_table)
